# Initial kernel scaffold; baseline (speedup 1.0000x reference)
import jax
import jax.numpy as jnp
from jax.experimental import pallas as pl


def kernel(x, batch, target, W1, as1, ad1, b1, pa, g1, be1, W2, as2, ad2, b2, cW, cb):
    raise NotImplementedError("write your pallas kernel here")


# speedup vs baseline: 1.0935x; 1.0935x over previous
import functools

import jax
import jax.numpy as jnp
from jax.experimental import pallas as pl
from jax.experimental.pallas import tpu as pltpu

N = 10000
FIN = 128
K = 32
NCLS = 40
H1, F1 = 4, 32
H2, F2 = 8, 16
NEG = 0.2

_BIG = 1e30
_IBIG = 2**30

_BR = 128
_NP = 8
_PW = 1280


def _d2_kernel(xr_ref, xc_ref, br_ref, bc_ref, out_ref):
    i = pl.program_id(0)
    xr = xr_ref[...]
    sqr = jnp.sum(xr * xr, axis=1, keepdims=True)
    batchr = br_ref[...][:, 0:1]
    npad = _NP * _PW
    rowid = i * _BR + jax.lax.broadcasted_iota(jnp.int32, (_BR, npad), 0)
    colid = jax.lax.broadcasted_iota(jnp.int32, (_BR, npad), 1)
    xc = xc_ref[...]
    mm = jax.lax.dot_general(
        xr, xc, (((1,), (1,)), ((), ())),
        preferred_element_type=jnp.float32)
    sqc = jnp.sum(xc * xc, axis=1)[None, :]
    d2 = sqr + sqc - 2.0 * mm
    batchc = bc_ref[0:1, :]
    bad = (batchr != batchc) | (colid == rowid)
    out_ref[...] = jnp.where(bad, _BIG, d2)


def _knn(x, batch, k):
    n, f = x.shape
    npad = _NP * _PW
    grid = npad // _BR
    x_pad = jnp.concatenate(
        [x, jnp.zeros((npad - n, f), x.dtype)], axis=0)
    b_pad = jnp.concatenate(
        [batch.astype(jnp.int32),
         jnp.full((npad - n,), jnp.int32(1 << 20))], axis=0)
    br2d = jnp.broadcast_to(b_pad[:, None], (npad, 8))
    bc2d = jnp.broadcast_to(b_pad[None, :], (8, npad))
    d2 = pl.pallas_call(
        _d2_kernel,
        grid=(grid,),
        in_specs=[
            pl.BlockSpec((_BR, f), lambda i: (i, 0)),
            pl.BlockSpec((npad, f), lambda i: (0, 0)),
            pl.BlockSpec((_BR, 8), lambda i: (i, 0)),
            pl.BlockSpec((8, npad), lambda i: (0, 0)),
        ],
        out_specs=pl.BlockSpec((_BR, npad), lambda i: (i, 0)),
        out_shape=jax.ShapeDtypeStruct((npad, npad), jnp.float32),
    )(x_pad, x_pad, br2d, bc2d)
    _, nbrs = jax.lax.top_k(-d2[:n, :n], k)
    return nbrs


def _leaky(v, s):
    return jnp.where(v >= 0, v, s * v)


def _gat(x, nbrs, W, a_src, a_dst, b, heads, fdim, concat, neg):
    n = x.shape[0]
    h = (x @ W).reshape(n, heads, fdim)
    al_s = jnp.sum(h * a_src[None], axis=-1)
    al_d = jnp.sum(h * a_dst[None], axis=-1)
    e = _leaky(al_s[nbrs] + al_d[:, None, :], neg)
    e = e - jnp.max(e, axis=1, keepdims=True)
    w = jnp.exp(e)
    w = w / jnp.sum(w, axis=1, keepdims=True)
    out = jnp.sum(w[..., None] * h[nbrs], axis=1)
    out = out.reshape(n, heads * fdim) if concat else jnp.mean(out, axis=1)
    return out + b


def _prelu_bn(x, a, g, b):
    y = jnp.where(x >= 0, x, a * x)
    m = jnp.mean(y, axis=0)
    v = jnp.var(y, axis=0)
    return (y - m) / jnp.sqrt(v + 1e-5) * g + b


def kernel(x, batch, target, W1, as1, ad1, b1, pa, g1, be1, W2, as2, ad2, b2,
           cW, cb):
    nbrs = _knn(x, batch, K)
    h = _gat(x, nbrs, W1, as1, ad1, b1, H1, F1, True, NEG)
    h = _prelu_bn(h, pa, g1, be1)
    nbrs2 = _knn(h, batch, K)
    h = _gat(h, nbrs2, W2, as2, ad2, b2, H2, F2, False, NEG)
    out = h @ cW + cb
    logp = jnp.log(out)
    loss = -jnp.mean(logp[jnp.arange(out.shape[0]), target])
    return (loss, out)


# speedup vs baseline: 1.0935x; 1.0000x over previous
import jax
import jax.numpy as jnp
from jax.experimental import pallas as pl

N = 10000
FIN = 128
K = 32
NCLS = 40
H1, F1 = 4, 32
H2, F2 = 8, 16
NEG = 0.2

_BIG = 1e30

_BR = 128
_NP = 8
_PW = 1280


def _d2_kernel(xr_ref, xc_ref, br_ref, bc_ref, out_ref):
    i = pl.program_id(0)
    xr = xr_ref[...]
    sqr = jnp.sum(xr * xr, axis=1, keepdims=True)
    batchr = br_ref[...][:, 0:1]
    npad = _NP * _PW
    rowid = i * _BR + jax.lax.broadcasted_iota(jnp.int32, (_BR, npad), 0)
    colid = jax.lax.broadcasted_iota(jnp.int32, (_BR, npad), 1)
    xc = xc_ref[...]
    mm = jax.lax.dot_general(
        xr, xc, (((1,), (1,)), ((), ())),
        preferred_element_type=jnp.float32)
    sqc = jnp.sum(xc * xc, axis=1)[None, :]
    d2 = sqr + sqc - 2.0 * mm
    batchc = bc_ref[0:1, :]
    bad = (batchr != batchc) | (colid == rowid)
    out_ref[...] = jnp.where(bad, _BIG, d2)


def _knn(x, batch, k):
    n, f = x.shape
    npad = _NP * _PW
    grid = npad // _BR
    x_pad = jnp.concatenate(
        [x, jnp.zeros((npad - n, f), x.dtype)], axis=0)
    b_pad = jnp.concatenate(
        [batch.astype(jnp.int32),
         jnp.full((npad - n,), jnp.int32(1 << 20))], axis=0)
    br2d = jnp.broadcast_to(b_pad[:, None], (npad, 8))
    bc2d = jnp.broadcast_to(b_pad[None, :], (8, npad))
    d2 = pl.pallas_call(
        _d2_kernel,
        grid=(grid,),
        in_specs=[
            pl.BlockSpec((_BR, f), lambda i: (i, 0)),
            pl.BlockSpec((npad, f), lambda i: (0, 0)),
            pl.BlockSpec((_BR, 8), lambda i: (i, 0)),
            pl.BlockSpec((8, npad), lambda i: (0, 0)),
        ],
        out_specs=pl.BlockSpec((_BR, npad), lambda i: (i, 0)),
        out_shape=jax.ShapeDtypeStruct((npad, npad), jnp.float32),
    )(x_pad, x_pad, br2d, bc2d)
    _, nbrs = jax.lax.top_k(-d2[:n, :n], k)
    return nbrs


def _leaky(v, s):
    return jnp.where(v >= 0, v, s * v)


def _gat(x, nbrs, W, a_src, a_dst, b, heads, fdim, concat, neg):
    n = x.shape[0]
    h = (x @ W).reshape(n, heads, fdim)
    al_s = jnp.sum(h * a_src[None], axis=-1)
    al_d = jnp.sum(h * a_dst[None], axis=-1)
    e = _leaky(al_s[nbrs] + al_d[:, None, :], neg)
    e = e - jnp.max(e, axis=1, keepdims=True)
    w = jnp.exp(e)
    w = w / jnp.sum(w, axis=1, keepdims=True)
    out = jnp.sum(w[..., None] * h[nbrs], axis=1)
    out = out.reshape(n, heads * fdim) if concat else jnp.mean(out, axis=1)
    return out + b


def _prelu_bn(x, a, g, b):
    y = jnp.where(x >= 0, x, a * x)
    m = jnp.mean(y, axis=0)
    v = jnp.var(y, axis=0)
    return (y - m) / jnp.sqrt(v + 1e-5) * g + b


def kernel(x, batch, target, W1, as1, ad1, b1, pa, g1, be1, W2, as2, ad2, b2,
           cW, cb):
    nbrs = _knn(x, batch, K)
    h = _gat(x, nbrs, W1, as1, ad1, b1, H1, F1, True, NEG)
    h = _prelu_bn(h, pa, g1, be1)
    nbrs2 = _knn(h, batch, K)
    h = _gat(h, nbrs2, W2, as2, ad2, b2, H2, F2, False, NEG)
    out = h @ cW + cb
    logp = jnp.log(out)
    loss = -jnp.mean(logp[jnp.arange(out.shape[0]), target])
    return (loss, out)
